# baseline (speedup 1.0000x reference)
import functools
import math

import jax
import jax.numpy as jnp
from jax import lax
from jax.experimental import pallas as pl
from jax.experimental.pallas import tpu as pltpu
from jax.experimental.pallas import tpu_sc as plsc

NUM_CORES = 2
NUM_SUBCORES = 16
NW = NUM_CORES * NUM_SUBCORES
LANES = 16
CHUNK = 16
NBUF = 8
DEPTH = 6


def _make_kernel(B, S, D, V):
    s_per_w = S // NW
    n_chunks = s_per_w // CHUNK
    n_units = n_chunks * B
    scale = jnp.float32(math.sqrt(float(D)))
    d_regs = D // LANES

    mesh = plsc.VectorSubcoreMesh(
        core_axis_name="c", subcore_axis_name="s",
        num_cores=NUM_CORES, num_subcores=NUM_SUBCORES)

    @functools.partial(
        pl.kernel,
        mesh=mesh,
        out_type=jax.ShapeDtypeStruct((B * S, D), jnp.float32),
        scratch_types=[
            pltpu.VMEM((B, s_per_w), jnp.int32),
            [pltpu.VMEM((CHUNK, D), jnp.float32) for _ in range(NBUF)],
            [pltpu.VMEM((CHUNK, D), jnp.float32) for _ in range(2)],
            pltpu.SemaphoreType.DMA,
            pltpu.SemaphoreType.DMA,
            pltpu.SemaphoreType.DMA,
        ],
    )
    def emb_kernel(x_hbm, wt_hbm, pos_hbm, out_hbm,
                   idx_v, wbufs, pbufs, gsem, ssem, psem):
        wid = lax.axis_index("s") * NUM_CORES + lax.axis_index("c")
        s_base = wid * s_per_w
        for b in range(B):
            pltpu.sync_copy(x_hbm.at[b, pl.ds(s_base, s_per_w)],
                            idx_v.at[b])

        def start_gather(u, buf):
            bb = lax.rem(u, B)
            ci = u // B
            pltpu.async_copy(
                wt_hbm.at[idx_v.at[bb, pl.ds(ci * CHUNK, CHUNK)]], buf, gsem)

        def wait_gather(buf):
            pltpu.make_async_copy(wt_hbm.at[pl.ds(0, CHUNK)], buf, gsem).wait()

        def start_pos(ci, buf):
            pltpu.async_copy(
                pos_hbm.at[pl.ds(s_base + ci * CHUNK, CHUNK)], buf, psem)

        def wait_pos(buf):
            pltpu.make_async_copy(
                pos_hbm.at[pl.ds(0, CHUNK)], buf, psem).wait()

        def start_store(u, buf):
            bb = lax.rem(u, B)
            ci = u // B
            row = bb * S + s_base + ci * CHUNK
            pltpu.async_copy(buf, out_hbm.at[pl.ds(row, CHUNK)], ssem)

        def wait_store(buf):
            pltpu.make_async_copy(buf, out_hbm.at[pl.ds(0, CHUNK)], ssem).wait()

        for k in range(DEPTH):
            start_gather(k, wbufs[k])
        start_pos(0, pbufs[0])

        def ring_body(up, _):
            for uu in range(NBUF):
                u = up * NBUF + uu
                wb = wbufs[uu]
                wb_ahead = wbufs[(uu + DEPTH) % NBUF]

                @pl.when(u < n_units - DEPTH)
                def _():
                    @pl.when(u >= NBUF - DEPTH)
                    def _():
                        wait_store(wb_ahead)
                    start_gather(u + DEPTH, wb_ahead)

                if uu % 4 == 0:
                    ci = up * 2 + uu // 4
                    pb_other = pbufs[1 - (uu // 4) % 2]

                    @pl.when(ci < n_chunks - 1)
                    def _():
                        start_pos(ci + 1, pb_other)

                wait_gather(wb)
                pb = pbufs[(uu // 4) % 2]
                if uu % 4 == 0:
                    wait_pos(pb)

                def row_body(r, _):
                    for j in range(d_regs):
                        sl = pl.ds(j * LANES, LANES)
                        wb[r, sl] = (wb[r, sl] + pb[r, sl]) * scale
                    return 0

                lax.fori_loop(0, CHUNK, row_body, 0)
                start_store(u, wb)
            return 0

        lax.fori_loop(0, n_units // NBUF, ring_body, 0)
        for q in range(NBUF):
            wait_store(wbufs[q])

    return emb_kernel


def kernel(x, word_table, pos_table):
    B, S = x.shape
    V, D = word_table.shape
    emb = _make_kernel(B, S, D, V)
    out = emb(x, word_table, pos_table[:S])
    return out.reshape(B, S, D)


# speedup vs baseline: 1.0201x; 1.0201x over previous
import functools
import math

import jax
import jax.numpy as jnp
from jax import lax
from jax.experimental import pallas as pl
from jax.experimental.pallas import tpu as pltpu
from jax.experimental.pallas import tpu_sc as plsc

NUM_CORES = 2
NUM_SUBCORES = 16
NW = NUM_CORES * NUM_SUBCORES
LANES = 16
CHUNK = 16
NBUF = 8
DEPTH = 5


def _make_kernel(B, S, D, V):
    s_per_w = S // NW
    n_chunks = s_per_w // CHUNK
    n_units = n_chunks * B
    scale = jnp.float32(math.sqrt(float(D)))
    d_regs = D // LANES

    mesh = plsc.VectorSubcoreMesh(
        core_axis_name="c", subcore_axis_name="s",
        num_cores=NUM_CORES, num_subcores=NUM_SUBCORES)

    @functools.partial(
        pl.kernel,
        mesh=mesh,
        out_type=jax.ShapeDtypeStruct((B * S, D), jnp.float32),
        scratch_types=[
            pltpu.VMEM((B * s_per_w,), jnp.int32),
            [pltpu.VMEM((CHUNK, D), jnp.float32) for _ in range(NBUF)],
            [pltpu.VMEM((CHUNK, D), jnp.float32) for _ in range(2)],
            pltpu.SemaphoreType.DMA,
            pltpu.SemaphoreType.DMA,
            pltpu.SemaphoreType.DMA,
        ],
    )
    def emb_kernel(x_hbm, wt_hbm, pos_hbm, out_hbm,
                   idx_v, wbufs, pbufs, gsem, ssem, psem):
        wid = lax.axis_index("s") * NUM_CORES + lax.axis_index("c")
        s_base = wid * s_per_w
        pltpu.sync_copy(x_hbm.at[wid], idx_v)

        def start_gather(u, buf):
            bb = lax.rem(u, B)
            ci = u // B
            pltpu.async_copy(
                wt_hbm.at[idx_v.at[pl.ds(bb * s_per_w + ci * CHUNK, CHUNK)]],
                buf, gsem)

        def wait_gather(buf):
            pltpu.make_async_copy(wt_hbm.at[pl.ds(0, CHUNK)], buf, gsem).wait()

        def start_pos(ci, buf):
            pltpu.async_copy(
                pos_hbm.at[pl.ds(s_base + ci * CHUNK, CHUNK)], buf, psem)

        def wait_pos(buf):
            pltpu.make_async_copy(
                pos_hbm.at[pl.ds(0, CHUNK)], buf, psem).wait()

        def start_store(u, buf):
            bb = lax.rem(u, B)
            ci = u // B
            row = bb * S + s_base + ci * CHUNK
            pltpu.async_copy(buf, out_hbm.at[pl.ds(row, CHUNK)], ssem)

        def wait_store(buf):
            pltpu.make_async_copy(buf, out_hbm.at[pl.ds(0, CHUNK)], ssem).wait()

        for k in range(DEPTH):
            start_gather(k, wbufs[k])
        start_pos(0, pbufs[0])

        def ring_body(up, _):
            for uu in range(NBUF):
                u = up * NBUF + uu
                wb = wbufs[uu]
                wb_ahead = wbufs[(uu + DEPTH) % NBUF]

                @pl.when(u < n_units - DEPTH)
                def _():
                    @pl.when(u >= NBUF - DEPTH)
                    def _():
                        wait_store(wb_ahead)
                    start_gather(u + DEPTH, wb_ahead)

                if uu % 4 == 0:
                    ci = up * 2 + uu // 4
                    pb_other = pbufs[1 - (uu // 4) % 2]

                    @pl.when(ci < n_chunks - 1)
                    def _():
                        start_pos(ci + 1, pb_other)

                wait_gather(wb)
                pb = pbufs[(uu // 4) % 2]
                if uu % 4 == 0:
                    wait_pos(pb)

                def row_body(r, _):
                    for j in range(d_regs):
                        sl = pl.ds(j * LANES, LANES)
                        wb[r, sl] = (wb[r, sl] + pb[r, sl]) * scale
                    return 0

                lax.fori_loop(0, CHUNK, row_body, 0)
                start_store(u, wb)
            return 0

        lax.fori_loop(0, n_units // NBUF, ring_body, 0)
        for q in range(NBUF):
            wait_store(wbufs[q])

    return emb_kernel


def kernel(x, word_table, pos_table):
    B, S = x.shape
    V, D = word_table.shape
    s_per_w = S // NW
    xw = x.reshape(B, NW, s_per_w).transpose(1, 0, 2).reshape(NW, B * s_per_w)
    emb = _make_kernel(B, S, D, V)
    out = emb(xw, word_table, pos_table[:S])
    return out.reshape(B, S, D)
